# baseline (speedup 1.0000x reference)
import math

import jax
import jax.numpy as jnp
from jax.experimental import pallas as pl
from jax.experimental.pallas import tpu as pltpu

_PADDING_IDX = 1
_SEQ_BLOCK = 256
_STRIDE = 16
_NQ = 8


def _body(x_ref, o_hbm, buf, sem):
    S = buf.shape[2]
    half = buf.shape[3] // 2
    nb = buf.shape[1]
    scale = math.log(10000.0) / (half - 1)
    i = pl.program_id(0)
    n = pl.num_programs(0)
    q = jax.lax.rem(i, _NQ)
    base = (i * S + _PADDING_IDX + 1).astype(jnp.float32)

    @pl.when(i >= _NQ)
    def _():
        pltpu.make_async_copy(
            buf.at[q], o_hbm.at[:, pl.ds((i - _NQ) * S, S), :], sem.at[q]
        ).wait()

    cols = jax.lax.broadcasted_iota(jnp.int32, (_STRIDE, half), 1)
    freq = jnp.exp(cols.astype(jnp.float32) * (-scale))
    rot_c = jnp.cos(freq * float(_STRIDE))
    rot_s = jnp.sin(freq * float(_STRIDE))

    rows0 = jax.lax.broadcasted_iota(jnp.int32, (_STRIDE, half), 0)
    ang0 = (rows0.astype(jnp.float32) + base) * freq
    sin0 = jnp.sin(ang0)
    cos0 = jnp.cos(ang0)

    def step(k, carry):
        s_k, c_k = carry
        xs = x_ref[pl.ds(k * _STRIDE, _STRIDE), :]
        for b in range(nb):
            m = (xs[:, b:b + 1] != _PADDING_IDX)
            buf[q, b, pl.ds(k * _STRIDE, _STRIDE), :half] = jnp.where(m, s_k, 0.0)
            buf[q, b, pl.ds(k * _STRIDE, _STRIDE), half:] = jnp.where(m, c_k, 0.0)
        s_n = s_k * rot_c + c_k * rot_s
        c_n = c_k * rot_c - s_k * rot_s
        return (s_n, c_n)

    jax.lax.fori_loop(0, S // _STRIDE, step, (sin0, cos0))

    pltpu.make_async_copy(
        buf.at[q], o_hbm.at[:, pl.ds(i * S, S), :], sem.at[q]).start()

    @pl.when(i == n - 1)
    def _():
        for qq in range(_NQ):
            kk = n - _NQ + qq
            pltpu.make_async_copy(
                buf.at[kk % _NQ], o_hbm.at[:, pl.ds(kk * S, S), :],
                sem.at[kk % _NQ]).wait()


def kernel(x, weights):
    bsz, seq_len = x.shape
    embed_dim = weights.shape[1]
    S = _SEQ_BLOCK
    num_seq = seq_len // S
    xt = x.T
    out = pl.pallas_call(
        _body,
        grid=(num_seq,),
        in_specs=[pl.BlockSpec((S, bsz), lambda i: (i, 0))],
        out_specs=pl.BlockSpec(memory_space=pl.ANY),
        out_shape=jax.ShapeDtypeStruct((bsz, seq_len, embed_dim), weights.dtype),
        scratch_shapes=[
            pltpu.VMEM((_NQ, bsz, S, embed_dim), jnp.float32),
            pltpu.SemaphoreType.DMA((_NQ,)),
        ],
    )(xt)
    return jax.lax.stop_gradient(out)


# speedup vs baseline: 1.0367x; 1.0367x over previous
import math

import jax
import jax.numpy as jnp
from jax.experimental import pallas as pl
from jax.experimental.pallas import tpu as pltpu

_PADDING_IDX = 1
_SEQ_BLOCK = 512
_STRIDE = 16
_NQ = 6


def _body(x_ref, o_hbm, buf, sem):
    S = buf.shape[2]
    half = buf.shape[3] // 2
    nb = buf.shape[1]
    scale = math.log(10000.0) / (half - 1)
    i = pl.program_id(0)
    n = pl.num_programs(0)
    q = jax.lax.rem(i, _NQ)
    base = (i * S + _PADDING_IDX + 1).astype(jnp.float32)

    @pl.when(i >= _NQ)
    def _():
        pltpu.make_async_copy(
            buf.at[q], o_hbm.at[:, pl.ds((i - _NQ) * S, S), :], sem.at[q]
        ).wait()

    cols = jax.lax.broadcasted_iota(jnp.int32, (_STRIDE, half), 1)
    freq = jnp.exp(cols.astype(jnp.float32) * (-scale))
    rot_c = jnp.cos(freq * float(_STRIDE))
    rot_s = jnp.sin(freq * float(_STRIDE))

    rows0 = jax.lax.broadcasted_iota(jnp.int32, (_STRIDE, half), 0)
    ang0 = (rows0.astype(jnp.float32) + base) * freq
    sin0 = jnp.sin(ang0)
    cos0 = jnp.cos(ang0)

    def step(k, carry):
        s_k, c_k = carry
        xs = x_ref[pl.ds(k * _STRIDE, _STRIDE), :]
        for b in range(nb):
            m = (xs[:, b:b + 1] != _PADDING_IDX)
            buf[q, b, pl.ds(k * _STRIDE, _STRIDE), :half] = jnp.where(m, s_k, 0.0)
            buf[q, b, pl.ds(k * _STRIDE, _STRIDE), half:] = jnp.where(m, c_k, 0.0)
        s_n = s_k * rot_c + c_k * rot_s
        c_n = c_k * rot_c - s_k * rot_s
        return (s_n, c_n)

    jax.lax.fori_loop(0, S // _STRIDE, step, (sin0, cos0))

    pltpu.make_async_copy(
        buf.at[q], o_hbm.at[:, pl.ds(i * S, S), :], sem.at[q]).start()

    @pl.when(i == n - 1)
    def _():
        for qq in range(_NQ):
            kk = n - _NQ + qq
            pltpu.make_async_copy(
                buf.at[kk % _NQ], o_hbm.at[:, pl.ds(kk * S, S), :],
                sem.at[kk % _NQ]).wait()


def kernel(x, weights):
    bsz, seq_len = x.shape
    embed_dim = weights.shape[1]
    S = _SEQ_BLOCK
    num_seq = seq_len // S
    xt = x.T
    out = pl.pallas_call(
        _body,
        grid=(num_seq,),
        in_specs=[pl.BlockSpec((S, bsz), lambda i: (i, 0))],
        out_specs=pl.BlockSpec(memory_space=pl.ANY),
        out_shape=jax.ShapeDtypeStruct((bsz, seq_len, embed_dim), weights.dtype),
        scratch_shapes=[
            pltpu.VMEM((_NQ, bsz, S, embed_dim), jnp.float32),
            pltpu.SemaphoreType.DMA((_NQ,)),
        ],
    )(xt)
    return jax.lax.stop_gradient(out)
